# Initial kernel scaffold; baseline (speedup 1.0000x reference)
import jax
import jax.numpy as jnp
from jax.experimental import pallas as pl


def kernel(x, edge_index, edge_attr, We1, be1, We2, be2, W1, b1, W2, b2, eps):
    raise NotImplementedError("write your pallas kernel here")


# speedup vs baseline: 2.0722x; 2.0722x over previous
import functools

import jax
import jax.numpy as jnp
from jax import lax
from jax.experimental import pallas as pl
from jax.experimental.pallas import tpu as pltpu
from jax.experimental.pallas import tpu_sc as plsc

N = 10000
E = 160000
D_IN = 256
D_EMB = 512
H = D_IN // 2

NC = 2
NS = 16
L = 16

EPT = E // NS
CH = 80
NCH = EPT // CH
IB = 25
NB = NCH // IB
WT = 10


def _edge_mlp_body(ea_ref, we1_ref, be1_ref, we2_ref, be2_ref, out_ref):
    ea = ea_ref[...]
    h1 = jnp.dot(ea, we1_ref[...], preferred_element_type=jnp.float32)
    h1 = jnp.maximum(h1 + be1_ref[...], 0.0)
    e = jnp.dot(h1, we2_ref[...], preferred_element_type=jnp.float32)
    e = e + be2_ref[...]
    out_ref[0] = e[:, :H]
    out_ref[1] = e[:, H:]


def _edge_mlp(edge_attr, We1, be1, We2, be2, block_e=2000):
    grid = (E // block_e,)
    return pl.pallas_call(
        _edge_mlp_body,
        grid=grid,
        in_specs=[
            pl.BlockSpec((block_e, 4), lambda i: (i, 0)),
            pl.BlockSpec((4, D_IN), lambda i: (0, 0)),
            pl.BlockSpec((1, D_IN), lambda i: (0, 0)),
            pl.BlockSpec((D_IN, D_IN), lambda i: (0, 0)),
            pl.BlockSpec((1, D_IN), lambda i: (0, 0)),
        ],
        out_specs=pl.BlockSpec((NC, block_e, H), lambda i: (0, i, 0)),
        out_shape=jax.ShapeDtypeStruct((NC, E, H), jnp.float32),
    )(edge_attr, We1, be1.reshape(1, D_IN), We2, be2.reshape(1, D_IN))


def _sc_body(xs_hbm, e_hbm, src_hbm, dst_hbm, out_hbm,
             src_v, dst_v, xrow_v, erow_v, aggr_sh, sem):
    c = lax.axis_index("c")
    s = lax.axis_index("s")

    off = c * N
    zf = jnp.zeros((L,), jnp.float32)

    def _zrow(r, carry):
        for k in range(H // L):
            xrow_v[r, pl.ds(k * L, L)] = zf
        return carry

    lax.fori_loop(0, CH, _zrow, 0)
    for k in range(8):
        chunk_id = s + k * NS

        @pl.when(chunk_id < NCH)
        def _zero_chunk():
            pltpu.sync_copy(xrow_v, aggr_sh.at[pl.ds(chunk_id * CH, CH)])

    plsc.subcore_barrier()

    def _block(b, carry):
        pltpu.sync_copy(src_hbm.at[s, b], src_v)
        pltpu.sync_copy(dst_hbm.at[s, b], dst_v)

        def _adj(j, jc):
            for k in range(CH // L):
                sl = pl.ds(k * L, L)
                src_v[j, sl] = src_v[j, sl] + off
            return jc

        lax.fori_loop(0, IB, _adj, 0)

        def _chunk(j, jc):
            pltpu.async_copy(xs_hbm.at[src_v.at[j]], xrow_v, sem).wait()
            row0 = s * EPT + (b * IB + j) * CH
            pltpu.sync_copy(e_hbm.at[c, pl.ds(row0, CH)], erow_v)

            def _row(r, rc):
                for k in range(H // L):
                    sl = pl.ds(k * L, L)
                    xrow_v[r, sl] = jnp.maximum(
                        xrow_v[r, sl] + erow_v[r, sl], 0.0)
                return rc

            lax.fori_loop(0, CH, _row, 0)
            pltpu.sync_copy(xrow_v, aggr_sh.at[dst_v.at[j]], add=True)
            return jc

        lax.fori_loop(0, IB, _chunk, 0)
        return carry

    lax.fori_loop(0, NB, _block, 0)
    plsc.subcore_barrier()

    rows_per_wt = N // WT

    @pl.when(s < WT)
    def _write_phase():
        pltpu.sync_copy(aggr_sh.at[pl.ds(s * rows_per_wt, rows_per_wt)],
                        out_hbm.at[c, pl.ds(s * rows_per_wt, rows_per_wt)])


def _sc_gather_scatter(xs, e2, src4, dst4):
    mesh = plsc.VectorSubcoreMesh(core_axis_name="c", subcore_axis_name="s",
                                  num_cores=NC, num_subcores=NS)
    fn = pl.kernel(
        _sc_body,
        out_type=jax.ShapeDtypeStruct((NC, N, H), jnp.float32),
        mesh=mesh,
        scratch_types=[
            pltpu.VMEM((IB, CH), jnp.int32),
            pltpu.VMEM((IB, CH), jnp.int32),
            pltpu.VMEM((CH, H), jnp.float32),
            pltpu.VMEM((CH, H), jnp.float32),
            pltpu.VMEM_SHARED((N, H), jnp.float32),
            pltpu.SemaphoreType.DMA,
        ],
    )
    return fn(xs, e2, src4, dst4)


def _node_mlp_body(x_ref, a_ref, w1_ref, b1_ref, w2_ref, b2_ref, eps_ref,
                   out_ref):
    scale = 1.0 + eps_ref[0, 0]
    aggr = jnp.concatenate([a_ref[0], a_ref[1]], axis=1)
    h = scale * x_ref[...] + aggr
    m = jnp.dot(h, w1_ref[...], preferred_element_type=jnp.float32)
    m = jnp.maximum(m + b1_ref[...], 0.0)
    o = jnp.dot(m, w2_ref[...], preferred_element_type=jnp.float32)
    out_ref[...] = o + b2_ref[...]


def _node_mlp(x, aggr2, W1, b1, W2, b2, eps, block_n=2000):
    grid = (N // block_n,)
    return pl.pallas_call(
        _node_mlp_body,
        grid=grid,
        in_specs=[
            pl.BlockSpec((block_n, D_IN), lambda i: (i, 0)),
            pl.BlockSpec((NC, block_n, H), lambda i: (0, i, 0)),
            pl.BlockSpec((D_IN, D_EMB), lambda i: (0, 0)),
            pl.BlockSpec((1, D_EMB), lambda i: (0, 0)),
            pl.BlockSpec((D_EMB, D_EMB), lambda i: (0, 0)),
            pl.BlockSpec((1, D_EMB), lambda i: (0, 0)),
            pl.BlockSpec(memory_space=pltpu.SMEM),
        ],
        out_specs=pl.BlockSpec((block_n, D_EMB), lambda i: (i, 0)),
        out_shape=jax.ShapeDtypeStruct((N, D_EMB), jnp.float32),
    )(x, aggr2, W1, b1.reshape(1, D_EMB), W2, b2.reshape(1, D_EMB),
      eps.reshape(1, 1))


def kernel(x, edge_index, edge_attr, We1, be1, We2, be2, W1, b1, W2, b2, eps):
    src = edge_index[0].astype(jnp.int32)
    dst = edge_index[1].astype(jnp.int32)
    src4 = src.reshape(NS, NB, IB, CH)
    dst4 = dst.reshape(NS, NB, IB, CH)
    xs = jnp.concatenate([x[:, :H], x[:, H:]], axis=0)

    e2 = _edge_mlp(edge_attr, We1, be1, We2, be2)
    aggr2 = _sc_gather_scatter(xs, e2, src4, dst4)
    return _node_mlp(x, aggr2, W1, b1, W2, b2, eps)


# speedup vs baseline: 3.2345x; 1.5609x over previous
import functools

import jax
import jax.numpy as jnp
from jax import lax
from jax.experimental import pallas as pl
from jax.experimental.pallas import tpu as pltpu
from jax.experimental.pallas import tpu_sc as plsc

N = 10000
E = 160000
D_IN = 256
D_EMB = 512
H = D_IN // 2

NC = 2
NS = 16
L = 16

EPT = E // NS
CH = 80
NCH = EPT // CH
IB = 25
NB = NCH // IB
WT = 10


def _edge_mlp_body(ea_ref, we1_ref, be1_ref, we2_ref, be2_ref, out_ref):
    ea = ea_ref[...]
    h1 = jnp.dot(ea, we1_ref[...], preferred_element_type=jnp.float32)
    h1 = jnp.maximum(h1 + be1_ref[...], 0.0)
    e = jnp.dot(h1, we2_ref[...], preferred_element_type=jnp.float32)
    e = e + be2_ref[...]
    out_ref[0] = e[:, :H]
    out_ref[1] = e[:, H:]


def _edge_mlp(edge_attr, We1, be1, We2, be2, block_e=2000):
    grid = (E // block_e,)
    return pl.pallas_call(
        _edge_mlp_body,
        grid=grid,
        in_specs=[
            pl.BlockSpec((block_e, 4), lambda i: (i, 0)),
            pl.BlockSpec((4, D_IN), lambda i: (0, 0)),
            pl.BlockSpec((1, D_IN), lambda i: (0, 0)),
            pl.BlockSpec((D_IN, D_IN), lambda i: (0, 0)),
            pl.BlockSpec((1, D_IN), lambda i: (0, 0)),
        ],
        out_specs=pl.BlockSpec((NC, block_e, H), lambda i: (0, i, 0)),
        out_shape=jax.ShapeDtypeStruct((NC, E, H), jnp.float32),
    )(edge_attr, We1, be1.reshape(1, D_IN), We2, be2.reshape(1, D_IN))


def _sc_body(x0_hbm, x1_hbm, e_hbm, src_hbm, dst_hbm, out_hbm,
             src_v, dst_v, xb0, xb1, eb0, eb1, dcur,
             aggr_sh, semg0, seme0, semg1, seme1):
    c = lax.axis_index("c")
    s = lax.axis_index("s")
    zf = jnp.zeros((L,), jnp.float32)

    def _zrow(r, carry):
        for k in range(H // L):
            xb0[r, pl.ds(k * L, L)] = zf
        return carry

    lax.fori_loop(0, CH, _zrow, 0)
    for k in range(8):
        chunk_id = s + k * NS

        @pl.when(chunk_id < NCH)
        def _zero_chunk():
            pltpu.sync_copy(xb0, aggr_sh.at[pl.ds(chunk_id * CH, CH)])

    plsc.subcore_barrier()


    def _load_block(b):
        pltpu.sync_copy(src_hbm.at[s, b], src_v)
        pltpu.sync_copy(dst_hbm.at[s, b], dst_v)

    def _issue(q, xb, eb, semg, seme):
        r = q % IB
        idx = src_v.at[r]

        @pl.when(c == 0)
        def _g0():
            pltpu.async_copy(x0_hbm.at[idx], xb, semg)

        @pl.when(c == 1)
        def _g1():
            pltpu.async_copy(x1_hbm.at[idx], xb, semg)

        pltpu.async_copy(e_hbm.at[c, pl.ds(s * EPT + q * CH, CH)], eb, seme)

    def _wait(xb, eb, semg, seme):
        @pl.when(c == 0)
        def _w0():
            pltpu.make_async_copy(x0_hbm.at[src_v.at[0]], xb, semg).wait()

        @pl.when(c == 1)
        def _w1():
            pltpu.make_async_copy(x1_hbm.at[src_v.at[0]], xb, semg).wait()

        pltpu.make_async_copy(e_hbm.at[c, pl.ds(0, CH)], eb, seme).wait()

    def _snap_dst(q):
        r = q % IB
        for k in range(CH // L):
            sl = pl.ds(k * L, L)
            dcur[sl] = dst_v[r, sl]

    def _compute_scatter(xb, eb):
        def _rows(i, carry):
            for rr in range(2):
                for k in range(H // L):
                    sl = pl.ds(k * L, L)
                    xb[2 * i + rr, sl] = jnp.maximum(
                        xb[2 * i + rr, sl] + eb[2 * i + rr, sl], 0.0)
            return carry

        lax.fori_loop(0, CH // 2, _rows, 0)
        pltpu.sync_copy(xb, aggr_sh.at[dcur], add=True)

    def _maybe_block(q):
        @pl.when(q % IB == 0)
        def _lb():
            _load_block(q // IB)

    _load_block(0)
    _issue(0, xb0, eb0, semg0, seme0)

    def _pair(m, carry):
        q0 = 2 * m
        q1 = q0 + 1
        q2 = q0 + 2
        _wait(xb0, eb0, semg0, seme0)
        _snap_dst(q0)
        _maybe_block(q1)
        _issue(q1, xb1, eb1, semg1, seme1)
        _compute_scatter(xb0, eb0)
        _wait(xb1, eb1, semg1, seme1)
        _snap_dst(q1)
        _maybe_block(q2)
        _issue(q2, xb0, eb0, semg0, seme0)
        _compute_scatter(xb1, eb1)
        return carry

    lax.fori_loop(0, (NCH - 1) // 2, _pair, 0)

    _wait(xb0, eb0, semg0, seme0)
    _snap_dst(NCH - 1)
    _compute_scatter(xb0, eb0)

    plsc.subcore_barrier()

    rows_per_wt = N // WT

    @pl.when(s < WT)
    def _write_phase():
        pltpu.sync_copy(aggr_sh.at[pl.ds(s * rows_per_wt, rows_per_wt)],
                        out_hbm.at[c, pl.ds(s * rows_per_wt, rows_per_wt)])


def _sc_gather_scatter(x0, x1, e2, src4, dst4):
    mesh = plsc.VectorSubcoreMesh(core_axis_name="c", subcore_axis_name="s",
                                  num_cores=NC, num_subcores=NS)
    fn = pl.kernel(
        _sc_body,
        out_type=jax.ShapeDtypeStruct((NC, N, H), jnp.float32),
        mesh=mesh,
        scratch_types=[
            pltpu.VMEM((IB, CH), jnp.int32),
            pltpu.VMEM((IB, CH), jnp.int32),
            pltpu.VMEM((CH, H), jnp.float32),
            pltpu.VMEM((CH, H), jnp.float32),
            pltpu.VMEM((CH, H), jnp.float32),
            pltpu.VMEM((CH, H), jnp.float32),
            pltpu.VMEM((CH,), jnp.int32),
            pltpu.VMEM_SHARED((N, H), jnp.float32),
            pltpu.SemaphoreType.DMA,
            pltpu.SemaphoreType.DMA,
            pltpu.SemaphoreType.DMA,
            pltpu.SemaphoreType.DMA,
        ],
    )
    return fn(x0, x1, e2, src4, dst4)


def _node_mlp_body(x_ref, a_ref, w1_ref, b1_ref, w2_ref, b2_ref, eps_ref,
                   out_ref):
    scale = 1.0 + eps_ref[0, 0]
    aggr = jnp.concatenate([a_ref[0], a_ref[1]], axis=1)
    h = scale * x_ref[...] + aggr
    m = jnp.dot(h, w1_ref[...], preferred_element_type=jnp.float32)
    m = jnp.maximum(m + b1_ref[...], 0.0)
    o = jnp.dot(m, w2_ref[...], preferred_element_type=jnp.float32)
    out_ref[...] = o + b2_ref[...]


def _node_mlp(x, aggr2, W1, b1, W2, b2, eps, block_n=2000):
    grid = (N // block_n,)
    return pl.pallas_call(
        _node_mlp_body,
        grid=grid,
        in_specs=[
            pl.BlockSpec((block_n, D_IN), lambda i: (i, 0)),
            pl.BlockSpec((NC, block_n, H), lambda i: (0, i, 0)),
            pl.BlockSpec((D_IN, D_EMB), lambda i: (0, 0)),
            pl.BlockSpec((1, D_EMB), lambda i: (0, 0)),
            pl.BlockSpec((D_EMB, D_EMB), lambda i: (0, 0)),
            pl.BlockSpec((1, D_EMB), lambda i: (0, 0)),
            pl.BlockSpec(memory_space=pltpu.SMEM),
        ],
        out_specs=pl.BlockSpec((block_n, D_EMB), lambda i: (i, 0)),
        out_shape=jax.ShapeDtypeStruct((N, D_EMB), jnp.float32),
    )(x, aggr2, W1, b1.reshape(1, D_EMB), W2, b2.reshape(1, D_EMB),
      eps.reshape(1, 1))


def kernel(x, edge_index, edge_attr, We1, be1, We2, be2, W1, b1, W2, b2, eps):
    src = edge_index[0].astype(jnp.int32)
    dst = edge_index[1].astype(jnp.int32)
    src4 = src.reshape(NS, NB, IB, CH)
    dst4 = dst.reshape(NS, NB, IB, CH)
    x0 = x[:, :H]
    x1 = x[:, H:]

    e2 = _edge_mlp(edge_attr, We1, be1, We2, be2)
    aggr2 = _sc_gather_scatter(x0, x1, e2, src4, dst4)
    return _node_mlp(x, aggr2, W1, b1, W2, b2, eps)


# speedup vs baseline: 3.2367x; 1.0007x over previous
import functools

import jax
import jax.numpy as jnp
from jax import lax
from jax.experimental import pallas as pl
from jax.experimental.pallas import tpu as pltpu
from jax.experimental.pallas import tpu_sc as plsc

N = 10000
E = 160000
D_IN = 256
D_EMB = 512
H = D_IN // 2

NC = 2
NS = 16
L = 16

EPT = E // NS
CH = 80
NCH = EPT // CH
IB = 25
NB = NCH // IB
WT = 10


def _edge_mlp_body(ea_ref, we1_ref, be1_ref, we2_ref, be2_ref, out_ref):
    ea = ea_ref[...].astype(jnp.bfloat16)
    h1 = jnp.dot(ea, we1_ref[...].astype(jnp.bfloat16),
                 preferred_element_type=jnp.float32)
    h1 = jnp.maximum(h1 + be1_ref[...], 0.0).astype(jnp.bfloat16)
    e = jnp.dot(h1, we2_ref[...].astype(jnp.bfloat16),
                preferred_element_type=jnp.float32)
    e = e + be2_ref[...]
    out_ref[0] = e[:, :H]
    out_ref[1] = e[:, H:]


def _edge_mlp(edge_attr, We1, be1, We2, be2, block_e=2000):
    grid = (E // block_e,)
    return pl.pallas_call(
        _edge_mlp_body,
        grid=grid,
        in_specs=[
            pl.BlockSpec((block_e, 4), lambda i: (i, 0)),
            pl.BlockSpec((4, D_IN), lambda i: (0, 0)),
            pl.BlockSpec((1, D_IN), lambda i: (0, 0)),
            pl.BlockSpec((D_IN, D_IN), lambda i: (0, 0)),
            pl.BlockSpec((1, D_IN), lambda i: (0, 0)),
        ],
        out_specs=pl.BlockSpec((NC, block_e, H), lambda i: (0, i, 0)),
        out_shape=jax.ShapeDtypeStruct((NC, E, H), jnp.float32),
    )(edge_attr, We1, be1.reshape(1, D_IN), We2, be2.reshape(1, D_IN))


def _sc_body(x0_hbm, x1_hbm, e_hbm, src_hbm, dst_hbm, out_hbm,
             src_v, dst_v, xb0, xb1, eb0, eb1, dcur,
             aggr_sh, semg0, seme0, semg1, seme1):
    c = lax.axis_index("c")
    s = lax.axis_index("s")
    zf = jnp.zeros((L,), jnp.float32)

    def _zrow(r, carry):
        for k in range(H // L):
            xb0[r, pl.ds(k * L, L)] = zf
        return carry

    lax.fori_loop(0, CH, _zrow, 0)
    for k in range(8):
        chunk_id = s + k * NS

        @pl.when(chunk_id < NCH)
        def _zero_chunk():
            pltpu.sync_copy(xb0, aggr_sh.at[pl.ds(chunk_id * CH, CH)])

    plsc.subcore_barrier()


    def _load_block(b):
        pltpu.sync_copy(src_hbm.at[s, b], src_v)
        pltpu.sync_copy(dst_hbm.at[s, b], dst_v)

    def _issue(q, xb, eb, semg, seme):
        r = q % IB
        idx = src_v.at[r]

        @pl.when(c == 0)
        def _g0():
            pltpu.async_copy(x0_hbm.at[idx], xb, semg)

        @pl.when(c == 1)
        def _g1():
            pltpu.async_copy(x1_hbm.at[idx], xb, semg)

        pltpu.async_copy(e_hbm.at[c, pl.ds(s * EPT + q * CH, CH)], eb, seme)

    def _wait(xb, eb, semg, seme):
        @pl.when(c == 0)
        def _w0():
            pltpu.make_async_copy(x0_hbm.at[src_v.at[0]], xb, semg).wait()

        @pl.when(c == 1)
        def _w1():
            pltpu.make_async_copy(x1_hbm.at[src_v.at[0]], xb, semg).wait()

        pltpu.make_async_copy(e_hbm.at[c, pl.ds(0, CH)], eb, seme).wait()

    def _snap_dst(q):
        r = q % IB
        for k in range(CH // L):
            sl = pl.ds(k * L, L)
            dcur[sl] = dst_v[r, sl]

    def _compute_scatter(xb, eb):
        def _rows(i, carry):
            for rr in range(2):
                for k in range(H // L):
                    sl = pl.ds(k * L, L)
                    xb[2 * i + rr, sl] = jnp.maximum(
                        xb[2 * i + rr, sl] + eb[2 * i + rr, sl], 0.0)
            return carry

        lax.fori_loop(0, CH // 2, _rows, 0)
        pltpu.sync_copy(xb, aggr_sh.at[dcur], add=True)

    def _maybe_block(q):
        @pl.when(q % IB == 0)
        def _lb():
            _load_block(q // IB)

    _load_block(0)
    _issue(0, xb0, eb0, semg0, seme0)

    def _pair(m, carry):
        q0 = 2 * m
        q1 = q0 + 1
        q2 = q0 + 2
        _wait(xb0, eb0, semg0, seme0)
        _snap_dst(q0)
        _maybe_block(q1)
        _issue(q1, xb1, eb1, semg1, seme1)
        _compute_scatter(xb0, eb0)
        _wait(xb1, eb1, semg1, seme1)
        _snap_dst(q1)
        _maybe_block(q2)
        _issue(q2, xb0, eb0, semg0, seme0)
        _compute_scatter(xb1, eb1)
        return carry

    lax.fori_loop(0, (NCH - 1) // 2, _pair, 0)

    _wait(xb0, eb0, semg0, seme0)
    _snap_dst(NCH - 1)
    _compute_scatter(xb0, eb0)

    plsc.subcore_barrier()

    rows_per_wt = N // WT

    @pl.when(s < WT)
    def _write_phase():
        pltpu.sync_copy(aggr_sh.at[pl.ds(s * rows_per_wt, rows_per_wt)],
                        out_hbm.at[c, pl.ds(s * rows_per_wt, rows_per_wt)])


def _sc_gather_scatter(x0, x1, e2, src4, dst4):
    mesh = plsc.VectorSubcoreMesh(core_axis_name="c", subcore_axis_name="s",
                                  num_cores=NC, num_subcores=NS)
    fn = pl.kernel(
        _sc_body,
        out_type=jax.ShapeDtypeStruct((NC, N, H), jnp.float32),
        mesh=mesh,
        scratch_types=[
            pltpu.VMEM((IB, CH), jnp.int32),
            pltpu.VMEM((IB, CH), jnp.int32),
            pltpu.VMEM((CH, H), jnp.float32),
            pltpu.VMEM((CH, H), jnp.float32),
            pltpu.VMEM((CH, H), jnp.float32),
            pltpu.VMEM((CH, H), jnp.float32),
            pltpu.VMEM((CH,), jnp.int32),
            pltpu.VMEM_SHARED((N, H), jnp.float32),
            pltpu.SemaphoreType.DMA,
            pltpu.SemaphoreType.DMA,
            pltpu.SemaphoreType.DMA,
            pltpu.SemaphoreType.DMA,
        ],
    )
    return fn(x0, x1, e2, src4, dst4)


def _node_mlp_body(x_ref, a_ref, w1_ref, b1_ref, w2_ref, b2_ref, eps_ref,
                   out_ref):
    scale = 1.0 + eps_ref[0, 0]
    aggr = jnp.concatenate([a_ref[0], a_ref[1]], axis=1)
    h = scale * x_ref[...] + aggr
    m = jnp.dot(h, w1_ref[...], preferred_element_type=jnp.float32)
    m = jnp.maximum(m + b1_ref[...], 0.0)
    o = jnp.dot(m, w2_ref[...], preferred_element_type=jnp.float32)
    out_ref[...] = o + b2_ref[...]


def _node_mlp(x, aggr2, W1, b1, W2, b2, eps, block_n=2000):
    grid = (N // block_n,)
    return pl.pallas_call(
        _node_mlp_body,
        grid=grid,
        in_specs=[
            pl.BlockSpec((block_n, D_IN), lambda i: (i, 0)),
            pl.BlockSpec((NC, block_n, H), lambda i: (0, i, 0)),
            pl.BlockSpec((D_IN, D_EMB), lambda i: (0, 0)),
            pl.BlockSpec((1, D_EMB), lambda i: (0, 0)),
            pl.BlockSpec((D_EMB, D_EMB), lambda i: (0, 0)),
            pl.BlockSpec((1, D_EMB), lambda i: (0, 0)),
            pl.BlockSpec(memory_space=pltpu.SMEM),
        ],
        out_specs=pl.BlockSpec((block_n, D_EMB), lambda i: (i, 0)),
        out_shape=jax.ShapeDtypeStruct((N, D_EMB), jnp.float32),
    )(x, aggr2, W1, b1.reshape(1, D_EMB), W2, b2.reshape(1, D_EMB),
      eps.reshape(1, 1))


def kernel(x, edge_index, edge_attr, We1, be1, We2, be2, W1, b1, W2, b2, eps):
    src = edge_index[0].astype(jnp.int32)
    dst = edge_index[1].astype(jnp.int32)
    src4 = src.reshape(NS, NB, IB, CH)
    dst4 = dst.reshape(NS, NB, IB, CH)
    x0 = x[:, :H]
    x1 = x[:, H:]

    e2 = _edge_mlp(edge_attr, We1, be1, We2, be2)
    aggr2 = _sc_gather_scatter(x0, x1, e2, src4, dst4)
    return _node_mlp(x, aggr2, W1, b1, W2, b2, eps)
